# baseline (speedup 1.0000x reference)
import functools

import jax
import jax.numpy as jnp
from jax import lax
from jax.experimental import pallas as pl
from jax.experimental.pallas import tpu as pltpu
from jax.experimental.pallas import tpu_sc as plsc

_E = 8
_K = 2
_B = 512
_NW = 32
_GC = 128
_CC = 128
_GATE_BT = 512
_LANES = 128


def _layernorm(h, g, b):
    mu = jnp.mean(h, axis=-1, keepdims=True)
    v = jnp.mean((h - mu) ** 2, axis=-1, keepdims=True)
    return (h - mu) * jax.lax.rsqrt(v + 1e-5) * g + b


def _gelu(h):
    return h * 0.5 * (1.0 + jax.lax.erf(h * 0.7071067811865476))


def _round_bf16_bits(xf):
    u = jax.lax.bitcast_convert_type(xf, jnp.int32)
    return u + jnp.int32(0x7FFF) + jax.lax.bitwise_and(
        jax.lax.shift_right_logical(u, jnp.int32(16)), jnp.int32(1))


def _gate_body(x_ref, gw1_ref, gb1_ref, glg_ref, glb_ref, gw2_ref, gb2_ref,
               gw_ref, tki_ref, tkw_ref, usage_ref, xi0_ref, *, n_tokens):
    xb = x_ref[...]
    half = xb.shape[1] // 2
    lo = jax.lax.shift_right_logical(_round_bf16_bits(xb[:, :half]), jnp.int32(16))
    hi = jax.lax.bitwise_and(_round_bf16_bits(xb[:, half:]), jnp.int32(-65536))
    xi0_ref[...] = jax.lax.bitwise_or(lo, hi)
    h = jnp.dot(xb, gw1_ref[...], preferred_element_type=jnp.float32) + gb1_ref[...]
    h = _gelu(_layernorm(h, glg_ref[...], glb_ref[...]))
    logits = jnp.dot(h, gw2_ref[...], preferred_element_type=jnp.float32) + gb2_ref[...]
    m = jnp.max(logits, axis=-1, keepdims=True)
    ex = jnp.exp(logits - m)
    gate_w = ex / jnp.sum(ex, axis=-1, keepdims=True)
    gw_ref[...] = gate_w[:, :_E]

    lane = jax.lax.broadcasted_iota(jnp.int32, gate_w.shape, 1)
    w0 = jnp.max(gate_w, axis=-1, keepdims=True)
    i0 = jnp.min(jnp.where(gate_w == w0, lane, _LANES), axis=-1, keepdims=True)
    masked = jnp.where(lane == i0, -1.0, gate_w)
    masked = jnp.where(lane >= _E, -1.0, masked)
    w1 = jnp.max(masked, axis=-1, keepdims=True)
    i1 = jnp.min(jnp.where(masked == w1, lane, _LANES), axis=-1, keepdims=True)
    s = w0 + w1
    tki_ref[...] = jnp.concatenate([i0, i1], axis=-1)
    tkw_ref[...] = jnp.concatenate([w0 / s, w1 / s], axis=-1)

    @pl.when(pl.program_id(0) == 0)
    def _():
        usage_ref[...] = jnp.zeros_like(usage_ref)

    usage_ref[...] += jnp.sum(gate_w[:, :_E], axis=0, keepdims=True) * (1.0 / n_tokens)


def _run_gate(x, gw1, gb1, glg, glb, gw2, gb2):
    n, d = x.shape
    h_dim = gw1.shape[1]
    bt = _GATE_BT
    gw2p = jnp.zeros((h_dim, _LANES), jnp.float32).at[:, :_E].set(gw2)
    gb2p = jnp.full((1, _LANES), -1e30, jnp.float32).at[0, :_E].set(gb2)
    full = lambda shape: pl.BlockSpec(shape, lambda i: (0,) * len(shape))
    return pl.pallas_call(
        functools.partial(_gate_body, n_tokens=n),
        grid=(n // bt,),
        in_specs=[
            pl.BlockSpec((bt, d), lambda i: (i, 0)),
            full((d, h_dim)),
            full((1, h_dim)), full((1, h_dim)), full((1, h_dim)),
            full((h_dim, _LANES)), full((1, _LANES)),
        ],
        out_specs=(
            pl.BlockSpec((bt, _E), lambda i: (i, 0)),
            pl.BlockSpec((bt, _K), lambda i: (i, 0)),
            pl.BlockSpec((bt, _K), lambda i: (i, 0)),
            full((1, _E)),
            pl.BlockSpec((bt, d // 2), lambda i: (i, 0)),
        ),
        out_shape=(
            jax.ShapeDtypeStruct((n, _E), jnp.float32),
            jax.ShapeDtypeStruct((n, _K), jnp.int32),
            jax.ShapeDtypeStruct((n, _K), jnp.float32),
            jax.ShapeDtypeStruct((1, _E), jnp.float32),
            jax.ShapeDtypeStruct((n, d // 2), jnp.int32),
        ),
        compiler_params=pltpu.CompilerParams(dimension_semantics=("arbitrary",)),
    )(x, gw1, gb1.reshape(1, h_dim), glg.reshape(1, h_dim), glb.reshape(1, h_dim),
      gw2p, gb2p)


def _routing(tki, tkw, n, m_slots):
    p = n * _K
    tki_flat = tki.reshape(p)
    onehot = (tki_flat[:, None] == jnp.arange(_E, dtype=jnp.int32)[None, :]).astype(jnp.int32)
    csum = jnp.cumsum(onehot, axis=0)
    counts = csum[-1]
    rank = jnp.sum(onehot * csum, axis=1) - 1
    aligned = ((counts + _B - 1) // _B) * _B
    ends = jnp.cumsum(aligned)
    starts = ends - aligned
    slot = jnp.sum(onehot * starts[None, :], axis=1) + rank
    pair_rows = (jnp.arange(p, dtype=jnp.int32) // _K)
    slot_rows = jnp.zeros((m_slots,), jnp.int32).at[slot].set(pair_rows)
    slot_w = jnp.zeros((m_slots,), jnp.float32).at[slot].set(tkw.reshape(p))
    nb = m_slots // _B
    block_expert = jnp.sum(
        (jnp.arange(nb, dtype=jnp.int32)[:, None] * _B >= ends[None, :]).astype(jnp.int32),
        axis=1)
    block_expert = jnp.minimum(block_expert, _E - 1)
    pos = slot.reshape(n, _K)
    return slot_rows, slot_w, block_expert, pos


def _sc_gather_body(xi_hbm, rows_hbm, xs_hbm, idx_a, idx_b, buf_a, buf_b,
                    is_a, is_b, gs_a, gs_b, ws_a, ws_b):
    wid = lax.axis_index("s") * 2 + lax.axis_index("c")
    rows_per_w = xs_hbm.shape[0] // _NW
    nch = rows_per_w // _GC
    base = wid * rows_per_w
    idxs, bufs = (idx_a, idx_b), (buf_a, buf_b)
    isems, gsems, wsems = (is_a, is_b), (gs_a, gs_b), (ws_a, ws_b)

    def issue_i(c, slot):
        return pltpu.async_copy(rows_hbm.at[pl.ds(base + c * _GC, _GC)],
                                idxs[slot], isems[slot])

    def issue_g(c, slot):
        return pltpu.async_copy(xi_hbm.at[idxs[slot]], bufs[slot], gsems[slot])

    def issue_w(c, slot):
        return pltpu.async_copy(bufs[slot],
                                xs_hbm.at[pl.ds(base + c * _GC, _GC)],
                                wsems[slot])

    gd = [None] * nch
    wd = [None] * nch
    issue_i(0, 0).wait()
    gd[0] = issue_g(0, 0)
    if nch > 1:
        issue_i(1, 1).wait()
    for c in range(nch):
        slot = c % 2
        nslot = (c + 1) % 2
        if c + 1 < nch:
            if c >= 1:
                wd[c - 1].wait()
            gd[c + 1] = issue_g(c + 1, nslot)
        gd[c].wait()
        wd[c] = issue_w(c, slot)
        if c + 2 < nch:
            issue_i(c + 2, slot).wait()
    if nch >= 2:
        wd[nch - 2].wait()
    wd[nch - 1].wait()


def _run_sc_gather(xi, slot_rows, m_slots):
    dw = xi.shape[1]
    mesh = plsc.VectorSubcoreMesh(core_axis_name="c", subcore_axis_name="s")
    return pl.kernel(
        _sc_gather_body,
        out_type=jax.ShapeDtypeStruct((m_slots, dw), jnp.int32),
        mesh=mesh,
        scratch_types=[
            pltpu.VMEM((_GC,), jnp.int32),
            pltpu.VMEM((_GC,), jnp.int32),
            pltpu.VMEM((_GC, dw), jnp.int32),
            pltpu.VMEM((_GC, dw), jnp.int32),
            pltpu.SemaphoreType.DMA,
            pltpu.SemaphoreType.DMA,
            pltpu.SemaphoreType.DMA,
            pltpu.SemaphoreType.DMA,
            pltpu.SemaphoreType.DMA,
            pltpu.SemaphoreType.DMA,
        ],
    )(xi, slot_rows)


def _experts_body(be_ref, xs_ref, ew1_ref, eb1_ref,
                  el1g_ref, el1b_ref, ew2_ref, eb2_ref, el2g_ref, el2b_ref,
                  ew3_ref, eb3_ref, opw_ref, opb_ref, w_ref, ys_ref):
    del be_ref
    xi = xs_ref[...]
    half = xi.shape[1]
    x_lo = jax.lax.bitcast_convert_type(
        jax.lax.shift_left(xi, jnp.int32(16)), jnp.float32)
    x_hi = jax.lax.bitcast_convert_type(
        jax.lax.bitwise_and(xi, jnp.int32(-65536)), jnp.float32)
    w1 = ew1_ref[0]
    h1 = (jnp.dot(x_lo, w1[:half], preferred_element_type=jnp.float32)
          + jnp.dot(x_hi, w1[half:], preferred_element_type=jnp.float32)
          + eb1_ref[0])
    h1 = _gelu(_layernorm(h1, el1g_ref[0], el1b_ref[0]))
    h2 = jnp.dot(h1, ew2_ref[0], preferred_element_type=jnp.float32) + eb2_ref[0]
    h2 = _gelu(_layernorm(h2, el2g_ref[0], el2b_ref[0]))
    eo = jnp.dot(h2, ew3_ref[0], preferred_element_type=jnp.float32) + eb3_ref[0]
    eo = jnp.dot(eo, opw_ref[...], preferred_element_type=jnp.float32) + opb_ref[...]
    ys_ref[...] = eo * w_ref[...]


def _run_experts(xs, block_expert, slot_w, ew1, eb1, el1g, el1b, ew2,
                 eb2, el2g, el2b, ew3, eb3, opw, opb, m_slots):
    d = ew1.shape[1]
    h_dim = ew1.shape[2]
    o_dim = ew3.shape[2]
    opw = jnp.zeros((o_dim, _LANES), jnp.float32).at[:, :o_dim].set(opw)
    opb = jnp.zeros((_LANES,), jnp.float32).at[:o_dim].set(opb)
    nb = m_slots // _B
    ex = lambda shape: pl.BlockSpec(shape, lambda i, be: (be[i],) + (0,) * (len(shape) - 1))
    grid_spec = pltpu.PrefetchScalarGridSpec(
        num_scalar_prefetch=1,
        grid=(nb,),
        in_specs=[
            pl.BlockSpec((_B, d // 2), lambda i, be: (i, 0)),
            ex((1, d, h_dim)), ex((1, 1, h_dim)),
            ex((1, 1, h_dim)), ex((1, 1, h_dim)),
            ex((1, h_dim, h_dim)), ex((1, 1, h_dim)),
            ex((1, 1, h_dim)), ex((1, 1, h_dim)),
            ex((1, h_dim, o_dim)), ex((1, 1, o_dim)),
            pl.BlockSpec((o_dim, _LANES), lambda i, be: (0, 0)),
            pl.BlockSpec((1, _LANES), lambda i, be: (0, 0)),
            pl.BlockSpec((_B, 1), lambda i, be: (i, 0)),
        ],
        out_specs=pl.BlockSpec((_B, _LANES), lambda i, be: (i, 0)),
    )
    return pl.pallas_call(
        _experts_body,
        grid_spec=grid_spec,
        out_shape=jax.ShapeDtypeStruct((m_slots, _LANES), jnp.float32),
        compiler_params=pltpu.CompilerParams(dimension_semantics=("arbitrary",)),
    )(block_expert, xs,
      ew1, eb1.reshape(_E, 1, h_dim),
      el1g.reshape(_E, 1, h_dim), el1b.reshape(_E, 1, h_dim),
      ew2, eb2.reshape(_E, 1, h_dim),
      el2g.reshape(_E, 1, h_dim), el2b.reshape(_E, 1, h_dim),
      ew3, eb3.reshape(_E, 1, o_dim),
      opw, opb.reshape(1, _LANES),
      slot_w.reshape(m_slots, 1))


def _sc_combine_body(ys_hbm, p0_hbm, p1_hbm, out_hbm,
                     i0_v, i1_v, g0_v, g1_v, sem0, sem1):
    wid = lax.axis_index("s") * 2 + lax.axis_index("c")
    n = out_hbm.shape[0]
    per_w = n // _NW
    base = wid * per_w
    for c in range(per_w // _CC):
        off = base + c * _CC
        pltpu.sync_copy(p0_hbm.at[pl.ds(off, _CC)], i0_v)
        pltpu.sync_copy(p1_hbm.at[pl.ds(off, _CC)], i1_v)
        cp0 = pltpu.async_copy(ys_hbm.at[i0_v], g0_v, sem0)
        cp1 = pltpu.async_copy(ys_hbm.at[i1_v], g1_v, sem1)
        cp0.wait()
        cp1.wait()

        def add_row(r, carry):
            for j in range(_LANES // 16):
                sl = pl.ds(j * 16, 16)
                g0_v[r, sl] = g0_v[r, sl] + g1_v[r, sl]
            return carry

        lax.fori_loop(0, _CC, add_row, 0)
        pltpu.sync_copy(g0_v, out_hbm.at[pl.ds(off, _CC)])


def _run_sc_combine(ys, pos, n):
    p0 = pos[:, 0].astype(jnp.int32)
    p1 = pos[:, 1].astype(jnp.int32)
    mesh = plsc.VectorSubcoreMesh(core_axis_name="c", subcore_axis_name="s")
    return pl.kernel(
        _sc_combine_body,
        out_type=jax.ShapeDtypeStruct((n, _LANES), jnp.float32),
        mesh=mesh,
        scratch_types=[
            pltpu.VMEM((_CC,), jnp.int32),
            pltpu.VMEM((_CC,), jnp.int32),
            pltpu.VMEM((_CC, _LANES), jnp.float32),
            pltpu.VMEM((_CC, _LANES), jnp.float32),
            pltpu.SemaphoreType.DMA,
            pltpu.SemaphoreType.DMA,
        ],
    )(ys, p0, p1)


def kernel(x, gw1, gb1, glg, glb, gw2, gb2, ew1, eb1, el1g, el1b, ew2, eb2,
           el2g, el2b, ew3, eb3, opw, opb):
    n, d = x.shape
    o_dim = ew3.shape[2]
    p = n * _K
    m_slots = ((p + _E * (_B - 1)) + _B - 1) // _B * _B

    gate_w, tki, tkw, usage, xi = _run_gate(x, gw1, gb1, glg, glb, gw2, gb2)
    slot_rows, slot_w, block_expert, pos = _routing(tki, tkw, n, m_slots)
    xs = _run_sc_gather(xi, slot_rows, m_slots)
    ys = _run_experts(xs, block_expert, slot_w, ew1, eb1, el1g, el1b,
                      ew2, eb2, el2g, el2b, ew3, eb3, opw, opb, m_slots)
    out = _run_sc_combine(ys, pos, n)
    return (out[:, :o_dim], gate_w, tki, tkw, usage.reshape(_E))


# speedup vs baseline: 1.1349x; 1.1349x over previous
import functools

import jax
import jax.numpy as jnp
from jax import lax
from jax.experimental import pallas as pl
from jax.experimental.pallas import tpu as pltpu
from jax.experimental.pallas import tpu_sc as plsc

_E = 8
_K = 2
_B = 256
_NW = 32
_GC = 96
_CC = 128
_GATE_BT = 512
_LANES = 128


def _layernorm(h, g, b):
    mu = jnp.mean(h, axis=-1, keepdims=True)
    v = jnp.mean((h - mu) ** 2, axis=-1, keepdims=True)
    return (h - mu) * jax.lax.rsqrt(v + 1e-5) * g + b


def _gelu(h):
    return h * 0.5 * (1.0 + jax.lax.erf(h * 0.7071067811865476))


def _round_bf16_bits(xf):
    u = jax.lax.bitcast_convert_type(xf, jnp.int32)
    return u + jnp.int32(0x7FFF) + jax.lax.bitwise_and(
        jax.lax.shift_right_logical(u, jnp.int32(16)), jnp.int32(1))


def _gate_body(x_ref, gw1_ref, gb1_ref, glg_ref, glb_ref, gw2_ref, gb2_ref,
               gw_ref, tki_ref, tkw_ref, usage_ref, xi0_ref, *, n_tokens):
    xb = x_ref[...]
    half = xb.shape[1] // 2
    lo = jax.lax.shift_right_logical(_round_bf16_bits(xb[:, :half]), jnp.int32(16))
    hi = jax.lax.bitwise_and(_round_bf16_bits(xb[:, half:]), jnp.int32(-65536))
    xi0_ref[...] = jax.lax.bitwise_or(lo, hi)
    h = jnp.dot(xb, gw1_ref[...], preferred_element_type=jnp.float32) + gb1_ref[...]
    h = _gelu(_layernorm(h, glg_ref[...], glb_ref[...]))
    logits = jnp.dot(h, gw2_ref[...], preferred_element_type=jnp.float32) + gb2_ref[...]
    m = jnp.max(logits, axis=-1, keepdims=True)
    ex = jnp.exp(logits - m)
    gate_w = ex / jnp.sum(ex, axis=-1, keepdims=True)
    gw_ref[...] = gate_w[:, :_E]

    lane = jax.lax.broadcasted_iota(jnp.int32, gate_w.shape, 1)
    w0 = jnp.max(gate_w, axis=-1, keepdims=True)
    i0 = jnp.min(jnp.where(gate_w == w0, lane, _LANES), axis=-1, keepdims=True)
    masked = jnp.where(lane == i0, -1.0, gate_w)
    masked = jnp.where(lane >= _E, -1.0, masked)
    w1 = jnp.max(masked, axis=-1, keepdims=True)
    i1 = jnp.min(jnp.where(masked == w1, lane, _LANES), axis=-1, keepdims=True)
    s = w0 + w1
    tki_ref[...] = jnp.concatenate([i0, i1], axis=-1)
    tkw_ref[...] = jnp.concatenate([w0 / s, w1 / s], axis=-1)

    @pl.when(pl.program_id(0) == 0)
    def _():
        usage_ref[...] = jnp.zeros_like(usage_ref)

    usage_ref[...] += jnp.sum(gate_w[:, :_E], axis=0, keepdims=True) * (1.0 / n_tokens)


def _run_gate(x, gw1, gb1, glg, glb, gw2, gb2):
    n, d = x.shape
    h_dim = gw1.shape[1]
    bt = _GATE_BT
    gw2p = jnp.zeros((h_dim, _LANES), jnp.float32).at[:, :_E].set(gw2)
    gb2p = jnp.full((1, _LANES), -1e30, jnp.float32).at[0, :_E].set(gb2)
    full = lambda shape: pl.BlockSpec(shape, lambda i: (0,) * len(shape))
    return pl.pallas_call(
        functools.partial(_gate_body, n_tokens=n),
        grid=(n // bt,),
        in_specs=[
            pl.BlockSpec((bt, d), lambda i: (i, 0)),
            full((d, h_dim)),
            full((1, h_dim)), full((1, h_dim)), full((1, h_dim)),
            full((h_dim, _LANES)), full((1, _LANES)),
        ],
        out_specs=(
            pl.BlockSpec((bt, _E), lambda i: (i, 0)),
            pl.BlockSpec((bt, _K), lambda i: (i, 0)),
            pl.BlockSpec((bt, _K), lambda i: (i, 0)),
            full((1, _E)),
            pl.BlockSpec((bt, d // 2), lambda i: (i, 0)),
        ),
        out_shape=(
            jax.ShapeDtypeStruct((n, _E), jnp.float32),
            jax.ShapeDtypeStruct((n, _K), jnp.int32),
            jax.ShapeDtypeStruct((n, _K), jnp.float32),
            jax.ShapeDtypeStruct((1, _E), jnp.float32),
            jax.ShapeDtypeStruct((n, d // 2), jnp.int32),
        ),
        compiler_params=pltpu.CompilerParams(dimension_semantics=("arbitrary",)),
    )(x, gw1, gb1.reshape(1, h_dim), glg.reshape(1, h_dim), glb.reshape(1, h_dim),
      gw2p, gb2p)


def _routing(tki, tkw, n, m_slots):
    p = n * _K
    tki_flat = tki.reshape(p)
    onehot = (tki_flat[:, None] == jnp.arange(_E, dtype=jnp.int32)[None, :]).astype(jnp.int32)
    csum = jnp.cumsum(onehot, axis=0)
    counts = csum[-1]
    rank = jnp.sum(onehot * csum, axis=1) - 1
    aligned = ((counts + _B - 1) // _B) * _B
    ends = jnp.cumsum(aligned)
    starts = ends - aligned
    slot = jnp.sum(onehot * starts[None, :], axis=1) + rank
    pair_rows = (jnp.arange(p, dtype=jnp.int32) // _K)
    slot_rows = jnp.zeros((m_slots,), jnp.int32).at[slot].set(pair_rows)
    slot_w = jnp.zeros((m_slots,), jnp.float32).at[slot].set(tkw.reshape(p))
    nb = m_slots // _B
    block_expert = jnp.sum(
        (jnp.arange(nb, dtype=jnp.int32)[:, None] * _B >= ends[None, :]).astype(jnp.int32),
        axis=1)
    block_expert = jnp.minimum(block_expert, _E - 1)
    pos = slot.reshape(n, _K)
    return slot_rows, slot_w, block_expert, pos


def _sc_gather_body(xi_hbm, rows_hbm, xs_hbm, idx_a, idx_b, buf_a, buf_b,
                    is_a, is_b, gs_a, gs_b, ws_a, ws_b):
    wid = lax.axis_index("s") * 2 + lax.axis_index("c")
    rows_per_w = xs_hbm.shape[0] // _NW
    nch = rows_per_w // _GC
    base = wid * rows_per_w
    idxs, bufs = (idx_a, idx_b), (buf_a, buf_b)
    isems, gsems, wsems = (is_a, is_b), (gs_a, gs_b), (ws_a, ws_b)

    def issue_i(c, slot):
        return pltpu.async_copy(rows_hbm.at[pl.ds(base + c * _GC, _GC)],
                                idxs[slot], isems[slot])

    def issue_g(c, slot):
        return pltpu.async_copy(xi_hbm.at[idxs[slot]], bufs[slot], gsems[slot])

    def issue_w(c, slot):
        return pltpu.async_copy(bufs[slot],
                                xs_hbm.at[pl.ds(base + c * _GC, _GC)],
                                wsems[slot])

    gd = [None] * nch
    wd = [None] * nch
    issue_i(0, 0).wait()
    gd[0] = issue_g(0, 0)
    if nch > 1:
        issue_i(1, 1).wait()
    for c in range(nch):
        slot = c % 2
        nslot = (c + 1) % 2
        if c + 1 < nch:
            if c >= 1:
                wd[c - 1].wait()
            gd[c + 1] = issue_g(c + 1, nslot)
        gd[c].wait()
        wd[c] = issue_w(c, slot)
        if c + 2 < nch:
            issue_i(c + 2, slot).wait()
    if nch >= 2:
        wd[nch - 2].wait()
    wd[nch - 1].wait()


def _run_sc_gather(xi, slot_rows, m_slots):
    dw = xi.shape[1]
    mesh = plsc.VectorSubcoreMesh(core_axis_name="c", subcore_axis_name="s")
    return pl.kernel(
        _sc_gather_body,
        out_type=jax.ShapeDtypeStruct((m_slots, dw), jnp.int32),
        mesh=mesh,
        scratch_types=[
            pltpu.VMEM((_GC,), jnp.int32),
            pltpu.VMEM((_GC,), jnp.int32),
            pltpu.VMEM((_GC, dw), jnp.int32),
            pltpu.VMEM((_GC, dw), jnp.int32),
            pltpu.SemaphoreType.DMA,
            pltpu.SemaphoreType.DMA,
            pltpu.SemaphoreType.DMA,
            pltpu.SemaphoreType.DMA,
            pltpu.SemaphoreType.DMA,
            pltpu.SemaphoreType.DMA,
        ],
    )(xi, slot_rows)


def _experts_body(be_ref, xs_ref, ew1_ref, eb1_ref,
                  el1g_ref, el1b_ref, ew2_ref, eb2_ref, el2g_ref, el2b_ref,
                  ew3_ref, eb3_ref, opw_ref, opb_ref, w_ref, ys_ref):
    del be_ref
    xi = xs_ref[...]
    half = xi.shape[1]
    x_lo = jax.lax.bitcast_convert_type(
        jax.lax.shift_left(xi, jnp.int32(16)), jnp.float32)
    x_hi = jax.lax.bitcast_convert_type(
        jax.lax.bitwise_and(xi, jnp.int32(-65536)), jnp.float32)
    w1 = ew1_ref[0]
    h1 = (jnp.dot(x_lo, w1[:half], preferred_element_type=jnp.float32)
          + jnp.dot(x_hi, w1[half:], preferred_element_type=jnp.float32)
          + eb1_ref[0])
    h1 = _gelu(_layernorm(h1, el1g_ref[0], el1b_ref[0]))
    h2 = jnp.dot(h1, ew2_ref[0], preferred_element_type=jnp.float32) + eb2_ref[0]
    h2 = _gelu(_layernorm(h2, el2g_ref[0], el2b_ref[0]))
    eo = jnp.dot(h2, ew3_ref[0], preferred_element_type=jnp.float32) + eb3_ref[0]
    eo = jnp.dot(eo, opw_ref[...], preferred_element_type=jnp.float32) + opb_ref[...]
    ys_ref[...] = eo * w_ref[...]


def _run_experts(xs, block_expert, slot_w, ew1, eb1, el1g, el1b, ew2,
                 eb2, el2g, el2b, ew3, eb3, opw, opb, m_slots):
    d = ew1.shape[1]
    h_dim = ew1.shape[2]
    o_dim = ew3.shape[2]
    opw = jnp.zeros((o_dim, _LANES), jnp.float32).at[:, :o_dim].set(opw)
    opb = jnp.zeros((_LANES,), jnp.float32).at[:o_dim].set(opb)
    nb = m_slots // _B
    ex = lambda shape: pl.BlockSpec(shape, lambda i, be: (be[i],) + (0,) * (len(shape) - 1))
    grid_spec = pltpu.PrefetchScalarGridSpec(
        num_scalar_prefetch=1,
        grid=(nb,),
        in_specs=[
            pl.BlockSpec((_B, d // 2), lambda i, be: (i, 0)),
            ex((1, d, h_dim)), ex((1, 1, h_dim)),
            ex((1, 1, h_dim)), ex((1, 1, h_dim)),
            ex((1, h_dim, h_dim)), ex((1, 1, h_dim)),
            ex((1, 1, h_dim)), ex((1, 1, h_dim)),
            ex((1, h_dim, o_dim)), ex((1, 1, o_dim)),
            pl.BlockSpec((o_dim, _LANES), lambda i, be: (0, 0)),
            pl.BlockSpec((1, _LANES), lambda i, be: (0, 0)),
            pl.BlockSpec((_B, 1), lambda i, be: (i, 0)),
        ],
        out_specs=pl.BlockSpec((_B, _LANES), lambda i, be: (i, 0)),
    )
    return pl.pallas_call(
        _experts_body,
        grid_spec=grid_spec,
        out_shape=jax.ShapeDtypeStruct((m_slots, _LANES), jnp.float32),
        compiler_params=pltpu.CompilerParams(dimension_semantics=("arbitrary",)),
    )(block_expert, xs,
      ew1, eb1.reshape(_E, 1, h_dim),
      el1g.reshape(_E, 1, h_dim), el1b.reshape(_E, 1, h_dim),
      ew2, eb2.reshape(_E, 1, h_dim),
      el2g.reshape(_E, 1, h_dim), el2b.reshape(_E, 1, h_dim),
      ew3, eb3.reshape(_E, 1, o_dim),
      opw, opb.reshape(1, _LANES),
      slot_w.reshape(m_slots, 1))


def _sc_combine_body(ys_hbm, p0_hbm, p1_hbm, out_hbm,
                     i0_v, i1_v, g0_v, g1_v, sem0, sem1):
    wid = lax.axis_index("s") * 2 + lax.axis_index("c")
    n = out_hbm.shape[0]
    per_w = n // _NW
    base = wid * per_w
    for c in range(per_w // _CC):
        off = base + c * _CC
        pltpu.sync_copy(p0_hbm.at[pl.ds(off, _CC)], i0_v)
        pltpu.sync_copy(p1_hbm.at[pl.ds(off, _CC)], i1_v)
        cp0 = pltpu.async_copy(ys_hbm.at[i0_v], g0_v, sem0)
        cp1 = pltpu.async_copy(ys_hbm.at[i1_v], g1_v, sem1)
        cp0.wait()
        cp1.wait()

        def add_row(r, carry):
            for j in range(_LANES // 16):
                sl = pl.ds(j * 16, 16)
                g0_v[r, sl] = g0_v[r, sl] + g1_v[r, sl]
            return carry

        lax.fori_loop(0, _CC, add_row, 0)
        pltpu.sync_copy(g0_v, out_hbm.at[pl.ds(off, _CC)])


def _run_sc_combine(ys, pos, n):
    p0 = pos[:, 0].astype(jnp.int32)
    p1 = pos[:, 1].astype(jnp.int32)
    mesh = plsc.VectorSubcoreMesh(core_axis_name="c", subcore_axis_name="s")
    return pl.kernel(
        _sc_combine_body,
        out_type=jax.ShapeDtypeStruct((n, _LANES), jnp.float32),
        mesh=mesh,
        scratch_types=[
            pltpu.VMEM((_CC,), jnp.int32),
            pltpu.VMEM((_CC,), jnp.int32),
            pltpu.VMEM((_CC, _LANES), jnp.float32),
            pltpu.VMEM((_CC, _LANES), jnp.float32),
            pltpu.SemaphoreType.DMA,
            pltpu.SemaphoreType.DMA,
        ],
    )(ys, p0, p1)


def kernel(x, gw1, gb1, glg, glb, gw2, gb2, ew1, eb1, el1g, el1b, ew2, eb2,
           el2g, el2b, ew3, eb3, opw, opb):
    n, d = x.shape
    o_dim = ew3.shape[2]
    p = n * _K
    m_slots = ((p + _E * (_B - 1)) + _B - 1) // _B * _B

    gate_w, tki, tkw, usage, xi = _run_gate(x, gw1, gb1, glg, glb, gw2, gb2)
    slot_rows, slot_w, block_expert, pos = _routing(tki, tkw, n, m_slots)
    xs = _run_sc_gather(xi, slot_rows, m_slots)
    ys = _run_experts(xs, block_expert, slot_w, ew1, eb1, el1g, el1b,
                      ew2, eb2, el2g, el2b, ew3, eb3, opw, opb, m_slots)
    out = _run_sc_combine(ys, pos, n)
    return (out[:, :o_dim], gate_w, tki, tkw, usage.reshape(_E))
